# baseline (speedup 1.0000x reference)
import functools
import jax
import jax.numpy as jnp
from jax import lax
from jax.experimental import pallas as pl
from jax.experimental.pallas import tpu as pltpu
from jax.experimental.pallas import tpu_sc as plsc

N_NODES = 10000
N_EDGES = 320000
D = 128

NC = 2
NS = 16
NW = NC * NS

CHUNK = 128
NCHUNKS = 82
EPW = NCHUNKS * CHUNK
E_PAD = EPW * NW

N_PAD = 10240
ROWS_PER_TILE = N_PAD // NS

_mesh = plsc.VectorSubcoreMesh(core_axis_name="c", subcore_axis_name="s")


def _deg_body(ei_hbm, zeros_hbm, out_hbm, idx_v, ones_v, acc_sh, sem):
    c = lax.axis_index("c")
    s = lax.axis_index("s")
    wid = s * NC + c
    for i in range(CHUNK // 16):
        ones_v[pl.ds(16 * i, 16)] = jnp.full((16,), 1.0, jnp.float32)
    pltpu.sync_copy(zeros_hbm.at[pl.ds(s * ROWS_PER_TILE, ROWS_PER_TILE)],
                    acc_sh.at[pl.ds(s * ROWS_PER_TILE, ROWS_PER_TILE)])
    pltpu.sync_copy(ei_hbm.at[wid], idx_v)
    plsc.subcore_barrier()

    pltpu.async_copy(ones_v, acc_sh.at[idx_v.at[0, 1]], sem[0], add=True)
    pltpu.async_copy(ones_v, acc_sh.at[idx_v.at[1, 1]], sem[1], add=True)

    @pl.loop(2, NCHUNKS, step=2)
    def _(j):
        pltpu.make_async_copy(ones_v, acc_sh.at[idx_v.at[j, 1]], sem[0]).wait()
        pltpu.async_copy(ones_v, acc_sh.at[idx_v.at[j, 1]], sem[0], add=True)
        pltpu.make_async_copy(ones_v, acc_sh.at[idx_v.at[j + 1, 1]], sem[1]).wait()
        pltpu.async_copy(ones_v, acc_sh.at[idx_v.at[j + 1, 1]], sem[1], add=True)

    pltpu.make_async_copy(ones_v, acc_sh.at[idx_v.at[0, 1]], sem[0]).wait()
    pltpu.make_async_copy(ones_v, acc_sh.at[idx_v.at[1, 1]], sem[1]).wait()

    plsc.subcore_barrier()
    pltpu.sync_copy(acc_sh.at[pl.ds(s * ROWS_PER_TILE, ROWS_PER_TILE)],
                    out_hbm.at[c, pl.ds(s * ROWS_PER_TILE, ROWS_PER_TILE)])


def _deg_counts(edge_idx, zeros_1d):
    k = pl.kernel(
        _deg_body,
        out_type=jax.ShapeDtypeStruct((NC, N_PAD), jnp.float32),
        mesh=_mesh,
        scratch_types=[
            pltpu.VMEM((NCHUNKS + 2, 2, CHUNK), jnp.int32),
            pltpu.VMEM((CHUNK,), jnp.float32),
            pltpu.VMEM_SHARED((N_PAD,), jnp.float32),
            [pltpu.SemaphoreType.DMA for _ in range(2)],
        ],
    )
    return k(edge_idx, zeros_1d)


def _scat_body(y_hbm, ei_hbm, zeros_hbm, out_hbm,
               bufs, idxs, acc_sh, gs, ss, isem):
    c = lax.axis_index("c")
    s = lax.axis_index("s")
    wid = s * NC + c
    rows = pl.ds(s * ROWS_PER_TILE, ROWS_PER_TILE)

    @pl.when(c == 0)
    def _():
        pltpu.sync_copy(y_hbm.at[rows], acc_sh.at[rows])

    @pl.when(c == 1)
    def _():
        pltpu.sync_copy(zeros_hbm.at[rows], acc_sh.at[rows])

    def load_idx(j, q):
        pltpu.async_copy(ei_hbm.at[wid, j], idxs[q], isem[q])

    def wait_idx(q):
        pltpu.make_async_copy(ei_hbm.at[wid, 0], idxs[q], isem[q]).wait()

    def step(j, b, q, first):
        if not first:
            pltpu.make_async_copy(bufs[b], acc_sh.at[idxs[q].at[1]], ss[b]).wait()
        load_idx(j + 2, (q + 2) % 4)
        wait_idx(q)
        pltpu.async_copy(y_hbm.at[idxs[q].at[0]], bufs[b], gs[b]).wait()
        pltpu.async_copy(bufs[b], acc_sh.at[idxs[q].at[1]], ss[b], add=True)

    plsc.subcore_barrier()
    load_idx(0, 0)
    load_idx(1, 1)
    step(0, 0, 0, True)
    step(1, 1, 1, True)

    @pl.loop(2, NCHUNKS, step=4)
    def _(j):
        step(j, 0, 2, False)
        step(j + 1, 1, 3, False)
        step(j + 2, 0, 0, False)
        step(j + 3, 1, 1, False)

    pltpu.make_async_copy(bufs[0], acc_sh.at[idxs[0].at[1]], ss[0]).wait()
    pltpu.make_async_copy(bufs[1], acc_sh.at[idxs[1].at[1]], ss[1]).wait()
    wait_idx(NCHUNKS % 4)
    wait_idx((NCHUNKS + 1) % 4)

    plsc.subcore_barrier()
    pltpu.sync_copy(acc_sh.at[pl.ds(s * ROWS_PER_TILE, ROWS_PER_TILE)],
                    out_hbm.at[c, pl.ds(s * ROWS_PER_TILE, ROWS_PER_TILE)])


def _edge_scatter(y, edge_idx, zeros_2d):
    k = pl.kernel(
        _scat_body,
        out_type=jax.ShapeDtypeStruct((NC, N_PAD, D), jnp.float32),
        mesh=_mesh,
        scratch_types=[
            [pltpu.VMEM((CHUNK, D), jnp.float32) for _ in range(2)],
            [pltpu.VMEM((2, CHUNK), jnp.int32) for _ in range(4)],
            pltpu.VMEM_SHARED((N_PAD, D), jnp.float32),
            [pltpu.SemaphoreType.DMA for _ in range(2)],
            [pltpu.SemaphoreType.DMA for _ in range(2)],
            [pltpu.SemaphoreType.DMA for _ in range(4)],
        ],
    )
    return k(y, edge_idx, zeros_2d)


BLK = 512
GRID = N_PAD // BLK


def _t0_body(x_ref, w_ref, xw_ref):
    xw_ref[...] = jnp.dot(x_ref[...], w_ref[...],
                          preferred_element_type=jnp.float32)


def _tc_matmul(x_pad, W1):
    return pl.pallas_call(
        _t0_body,
        grid=(GRID,),
        in_specs=[
            pl.BlockSpec((BLK, D), lambda i: (i, 0)),
            pl.BlockSpec((D, D), lambda i: (0, 0)),
        ],
        out_specs=pl.BlockSpec((BLK, D), lambda i: (i, 0)),
        out_shape=jax.ShapeDtypeStruct((N_PAD, D), jnp.float32),
    )(x_pad, W1)


def _t1_body(ca_ref, cb_ref, xw_ref, y_ref, dinv_ref):
    deg = 1.0 + ca_ref[...] + cb_ref[...]
    dinv = lax.rsqrt(deg)
    dinv_ref[...] = dinv
    y_ref[...] = xw_ref[...] * dinv


def _tc_scale(cnt_a, cnt_b, xw):
    return pl.pallas_call(
        _t1_body,
        grid=(GRID,),
        in_specs=[
            pl.BlockSpec((BLK, 1), lambda i: (i, 0)),
            pl.BlockSpec((BLK, 1), lambda i: (i, 0)),
            pl.BlockSpec((BLK, D), lambda i: (i, 0)),
        ],
        out_specs=[
            pl.BlockSpec((BLK, D), lambda i: (i, 0)),
            pl.BlockSpec((BLK, 1), lambda i: (i, 0)),
        ],
        out_shape=[
            jax.ShapeDtypeStruct((N_PAD, D), jnp.float32),
            jax.ShapeDtypeStruct((N_PAD, 1), jnp.float32),
        ],
    )(cnt_a, cnt_b, xw)


def _t2_body(sa_ref, sb_ref, dinv_ref, b_ref, w_ref, y2_ref):
    dinv = dinv_ref[...]
    agg = sa_ref[...] + sb_ref[...]
    h = jnp.maximum(agg * dinv + b_ref[...], 0.0)
    y2_ref[...] = jnp.dot(h, w_ref[...], preferred_element_type=jnp.float32) * dinv


def _tc_mid(sa, sb, dinv, b, W):
    return pl.pallas_call(
        _t2_body,
        grid=(GRID,),
        in_specs=[
            pl.BlockSpec((BLK, D), lambda i: (i, 0)),
            pl.BlockSpec((BLK, D), lambda i: (i, 0)),
            pl.BlockSpec((BLK, 1), lambda i: (i, 0)),
            pl.BlockSpec((1, D), lambda i: (0, 0)),
            pl.BlockSpec((D, D), lambda i: (0, 0)),
        ],
        out_specs=pl.BlockSpec((BLK, D), lambda i: (i, 0)),
        out_shape=jax.ShapeDtypeStruct((N_PAD, D), jnp.float32),
    )(sa, sb, dinv, b, W)


def _t3_body(sa_ref, sb_ref, dinv_ref, b_ref, w_ref, bo_ref, o_ref):
    agg = sa_ref[...] + sb_ref[...]
    h = jnp.maximum(agg * dinv_ref[...] + b_ref[...], 0.0)
    o_ref[...] = jnp.dot(h, w_ref[...], preferred_element_type=jnp.float32) + bo_ref[...]


def _tc_last(sa, sb, dinv, b, W, bo):
    return pl.pallas_call(
        _t3_body,
        grid=(GRID,),
        in_specs=[
            pl.BlockSpec((BLK, D), lambda i: (i, 0)),
            pl.BlockSpec((BLK, D), lambda i: (i, 0)),
            pl.BlockSpec((BLK, 1), lambda i: (i, 0)),
            pl.BlockSpec((1, D), lambda i: (0, 0)),
            pl.BlockSpec((D, D), lambda i: (0, 0)),
            pl.BlockSpec((1, D), lambda i: (0, 0)),
        ],
        out_specs=pl.BlockSpec((BLK, D), lambda i: (i, 0)),
        out_shape=jax.ShapeDtypeStruct((N_PAD, D), jnp.float32),
    )(sa, sb, dinv, b, W, bo)


def kernel(x, edge_index, W1, b1, W2, b2, Wo, bo):
    src = edge_index[0].astype(jnp.int32)
    dst = edge_index[1].astype(jnp.int32)

    n_fill = E_PAD - N_EDGES
    fill = (N_NODES + (jnp.arange(n_fill, dtype=jnp.int32) % (N_PAD - N_NODES)))
    src_p = jnp.concatenate([src, fill]).reshape(NW, NCHUNKS, CHUNK)
    dst_p = jnp.concatenate([dst, fill]).reshape(NW, NCHUNKS, CHUNK)
    ei = jnp.stack([src_p, dst_p], axis=2)
    ei = jnp.concatenate(
        [ei, jnp.zeros((NW, 2, 2, CHUNK), jnp.int32)], axis=1)

    x_pad = jnp.zeros((N_PAD, D), jnp.float32).at[:N_NODES].set(x)
    zeros_1d = jnp.zeros((N_PAD,), jnp.float32)
    zeros_2d = jnp.zeros((N_PAD, D), jnp.float32)

    cnt = _deg_counts(ei, zeros_1d)
    xw1 = _tc_matmul(x_pad, W1)
    cnt_a = cnt[0].reshape(N_PAD, 1)
    cnt_b = cnt[1].reshape(N_PAD, 1)

    y1, dinv = _tc_scale(cnt_a, cnt_b, xw1)
    s1 = _edge_scatter(y1, ei, zeros_2d)
    y2 = _tc_mid(s1[0], s1[1], dinv, b1.reshape(1, D), W2)
    s2 = _edge_scatter(y2, ei, zeros_2d)
    out = _tc_last(s2[0], s2[1], dinv, b2.reshape(1, D), Wo, bo.reshape(1, D))
    return out[:N_NODES]


# speedup vs baseline: 1.0355x; 1.0355x over previous
import jax
import jax.numpy as jnp
from jax import lax
from jax.experimental import pallas as pl
from jax.experimental.pallas import tpu as pltpu
from jax.experimental.pallas import tpu_sc as plsc

N_NODES = 10000
N_EDGES = 320000
D = 128

NC = 2
NS = 16
NW = NC * NS

CHUNK = 128
NCHUNKS = 80
EPW = NCHUNKS * CHUNK
E_PAD = EPW * NW

N_PAD = 10240
ROWS_PER_TILE = N_PAD // NS

_mesh = plsc.VectorSubcoreMesh(core_axis_name="c", subcore_axis_name="s")


def _deg_body(ei_hbm, zeros_hbm, out_hbm, idx_v, ones_v, acc_sh, sem):
    c = lax.axis_index("c")
    s = lax.axis_index("s")
    wid = s * NC + c
    for i in range(CHUNK // 16):
        ones_v[pl.ds(16 * i, 16)] = jnp.full((16,), 1.0, jnp.float32)
    pltpu.sync_copy(zeros_hbm.at[pl.ds(s * ROWS_PER_TILE, ROWS_PER_TILE)],
                    acc_sh.at[pl.ds(s * ROWS_PER_TILE, ROWS_PER_TILE)])
    pltpu.sync_copy(ei_hbm.at[wid], idx_v)
    plsc.subcore_barrier()

    pltpu.async_copy(ones_v, acc_sh.at[idx_v.at[0, 1]], sem[0], add=True)
    pltpu.async_copy(ones_v, acc_sh.at[idx_v.at[1, 1]], sem[1], add=True)

    @pl.loop(2, NCHUNKS, step=2)
    def _(j):
        pltpu.make_async_copy(ones_v, acc_sh.at[idx_v.at[j, 1]], sem[0]).wait()
        pltpu.async_copy(ones_v, acc_sh.at[idx_v.at[j, 1]], sem[0], add=True)
        pltpu.make_async_copy(ones_v, acc_sh.at[idx_v.at[j + 1, 1]], sem[1]).wait()
        pltpu.async_copy(ones_v, acc_sh.at[idx_v.at[j + 1, 1]], sem[1], add=True)

    pltpu.make_async_copy(ones_v, acc_sh.at[idx_v.at[0, 1]], sem[0]).wait()
    pltpu.make_async_copy(ones_v, acc_sh.at[idx_v.at[1, 1]], sem[1]).wait()

    plsc.subcore_barrier()
    pltpu.sync_copy(acc_sh.at[pl.ds(s * ROWS_PER_TILE, ROWS_PER_TILE)],
                    out_hbm.at[c, pl.ds(s * ROWS_PER_TILE, ROWS_PER_TILE)])


def _deg_counts(edge_idx, zeros_1d):
    k = pl.kernel(
        _deg_body,
        out_type=jax.ShapeDtypeStruct((NC, N_PAD), jnp.float32),
        mesh=_mesh,
        scratch_types=[
            pltpu.VMEM((NCHUNKS + 2, 2, CHUNK), jnp.int32),
            pltpu.VMEM((CHUNK,), jnp.float32),
            pltpu.VMEM_SHARED((N_PAD,), jnp.float32),
            [pltpu.SemaphoreType.DMA for _ in range(2)],
        ],
    )
    return k(edge_idx, zeros_1d)


def _scat_body(y_hbm, ei_hbm, zeros_hbm, out_hbm,
               bufs, idxs, acc_sh, gs, ss, isem):
    c = lax.axis_index("c")
    s = lax.axis_index("s")
    wid = s * NC + c
    rows = pl.ds(s * ROWS_PER_TILE, ROWS_PER_TILE)

    @pl.when(c == 0)
    def _():
        pltpu.sync_copy(y_hbm.at[rows], acc_sh.at[rows])

    @pl.when(c == 1)
    def _():
        pltpu.sync_copy(zeros_hbm.at[rows], acc_sh.at[rows])

    def load_idx(j, q):
        pltpu.async_copy(ei_hbm.at[wid, j], idxs[q], isem[q])

    def wait_idx(q):
        pltpu.make_async_copy(ei_hbm.at[wid, 0], idxs[q], isem[q]).wait()

    def step(j, b, q, first):
        if not first:
            pltpu.make_async_copy(bufs[b], acc_sh.at[idxs[q].at[1]], ss[b]).wait()
        load_idx(j + 2, (q + 2) % 4)
        wait_idx(q)
        pltpu.async_copy(y_hbm.at[idxs[q].at[0]], bufs[b], gs[b]).wait()
        pltpu.async_copy(bufs[b], acc_sh.at[idxs[q].at[1]], ss[b], add=True)

    plsc.subcore_barrier()
    load_idx(0, 0)
    load_idx(1, 1)
    step(0, 0, 0, True)
    step(1, 1, 1, True)
    step(2, 0, 2, False)
    step(3, 1, 3, False)

    @pl.loop(4, NCHUNKS, step=4)
    def _(j):
        step(j, 0, 0, False)
        step(j + 1, 1, 1, False)
        step(j + 2, 0, 2, False)
        step(j + 3, 1, 3, False)

    pltpu.make_async_copy(bufs[0], acc_sh.at[idxs[0].at[1]], ss[0]).wait()
    pltpu.make_async_copy(bufs[1], acc_sh.at[idxs[1].at[1]], ss[1]).wait()
    wait_idx(NCHUNKS % 4)
    wait_idx((NCHUNKS + 1) % 4)

    plsc.subcore_barrier()
    pltpu.sync_copy(acc_sh.at[pl.ds(s * ROWS_PER_TILE, ROWS_PER_TILE)],
                    out_hbm.at[c, pl.ds(s * ROWS_PER_TILE, ROWS_PER_TILE)])


def _edge_scatter(y, edge_idx, zeros_2d):
    k = pl.kernel(
        _scat_body,
        out_type=jax.ShapeDtypeStruct((NC, N_PAD, D), jnp.float32),
        mesh=_mesh,
        scratch_types=[
            [pltpu.VMEM((CHUNK, D), jnp.float32) for _ in range(2)],
            [pltpu.VMEM((2, CHUNK), jnp.int32) for _ in range(4)],
            pltpu.VMEM_SHARED((N_PAD, D), jnp.float32),
            [pltpu.SemaphoreType.DMA for _ in range(2)],
            [pltpu.SemaphoreType.DMA for _ in range(2)],
            [pltpu.SemaphoreType.DMA for _ in range(4)],
        ],
    )
    return k(y, edge_idx, zeros_2d)


BLK = 512
GRID = N_PAD // BLK


def _t1_body(ca_ref, cb_ref, x_ref, w_ref, y_ref, dinv_ref):
    deg = 1.0 + ca_ref[...] + cb_ref[...]
    dinv = lax.rsqrt(deg)
    dinv_ref[...] = dinv
    xw = jnp.dot(x_ref[...], w_ref[...], preferred_element_type=jnp.float32)
    y_ref[...] = xw * dinv


def _tc_first(cnt_a, cnt_b, x_pad, W1):
    return pl.pallas_call(
        _t1_body,
        grid=(GRID,),
        in_specs=[
            pl.BlockSpec((BLK, 1), lambda i: (i, 0)),
            pl.BlockSpec((BLK, 1), lambda i: (i, 0)),
            pl.BlockSpec((BLK, D), lambda i: (i, 0)),
            pl.BlockSpec((D, D), lambda i: (0, 0)),
        ],
        out_specs=[
            pl.BlockSpec((BLK, D), lambda i: (i, 0)),
            pl.BlockSpec((BLK, 1), lambda i: (i, 0)),
        ],
        out_shape=[
            jax.ShapeDtypeStruct((N_PAD, D), jnp.float32),
            jax.ShapeDtypeStruct((N_PAD, 1), jnp.float32),
        ],
    )(cnt_a, cnt_b, x_pad, W1)


def _t2_body(sa_ref, sb_ref, dinv_ref, b_ref, w_ref, y2_ref):
    dinv = dinv_ref[...]
    agg = sa_ref[...] + sb_ref[...]
    h = jnp.maximum(agg * dinv + b_ref[...], 0.0)
    y2_ref[...] = jnp.dot(h, w_ref[...], preferred_element_type=jnp.float32) * dinv


def _tc_mid(sa, sb, dinv, b, W):
    return pl.pallas_call(
        _t2_body,
        grid=(GRID,),
        in_specs=[
            pl.BlockSpec((BLK, D), lambda i: (i, 0)),
            pl.BlockSpec((BLK, D), lambda i: (i, 0)),
            pl.BlockSpec((BLK, 1), lambda i: (i, 0)),
            pl.BlockSpec((1, D), lambda i: (0, 0)),
            pl.BlockSpec((D, D), lambda i: (0, 0)),
        ],
        out_specs=pl.BlockSpec((BLK, D), lambda i: (i, 0)),
        out_shape=jax.ShapeDtypeStruct((N_PAD, D), jnp.float32),
    )(sa, sb, dinv, b, W)


def _t3_body(sa_ref, sb_ref, dinv_ref, b_ref, w_ref, bo_ref, o_ref):
    agg = sa_ref[...] + sb_ref[...]
    h = jnp.maximum(agg * dinv_ref[...] + b_ref[...], 0.0)
    o_ref[...] = jnp.dot(h, w_ref[...], preferred_element_type=jnp.float32) + bo_ref[...]


def _tc_last(sa, sb, dinv, b, W, bo):
    return pl.pallas_call(
        _t3_body,
        grid=(GRID,),
        in_specs=[
            pl.BlockSpec((BLK, D), lambda i: (i, 0)),
            pl.BlockSpec((BLK, D), lambda i: (i, 0)),
            pl.BlockSpec((BLK, 1), lambda i: (i, 0)),
            pl.BlockSpec((1, D), lambda i: (0, 0)),
            pl.BlockSpec((D, D), lambda i: (0, 0)),
            pl.BlockSpec((1, D), lambda i: (0, 0)),
        ],
        out_specs=pl.BlockSpec((BLK, D), lambda i: (i, 0)),
        out_shape=jax.ShapeDtypeStruct((N_PAD, D), jnp.float32),
    )(sa, sb, dinv, b, W, bo)


def kernel(x, edge_index, W1, b1, W2, b2, Wo, bo):
    src = edge_index[0].astype(jnp.int32)
    dst = edge_index[1].astype(jnp.int32)

    n_fill = E_PAD - N_EDGES
    fill = (N_NODES + (jnp.arange(n_fill, dtype=jnp.int32) % (N_PAD - N_NODES)))
    src_p = jnp.concatenate([src, fill]).reshape(NW, NCHUNKS, CHUNK)
    dst_p = jnp.concatenate([dst, fill]).reshape(NW, NCHUNKS, CHUNK)
    ei = jnp.stack([src_p, dst_p], axis=2)
    ei = jnp.concatenate(
        [ei, jnp.zeros((NW, 2, 2, CHUNK), jnp.int32)], axis=1)

    x_pad = jnp.zeros((N_PAD, D), jnp.float32).at[:N_NODES].set(x)
    zeros_1d = jnp.zeros((N_PAD,), jnp.float32)
    zeros_2d = jnp.zeros((N_PAD, D), jnp.float32)

    cnt = _deg_counts(ei, zeros_1d)
    cnt_a = cnt[0].reshape(N_PAD, 1)
    cnt_b = cnt[1].reshape(N_PAD, 1)

    y1, dinv = _tc_first(cnt_a, cnt_b, x_pad, W1)
    s1 = _edge_scatter(y1, ei, zeros_2d)
    y2 = _tc_mid(s1[0], s1[1], dinv, b1.reshape(1, D), W2)
    s2 = _edge_scatter(y2, ei, zeros_2d)
    out = _tc_last(s2[0], s2[1], dinv, b2.reshape(1, D), Wo, bo.reshape(1, D))
    return out[:N_NODES]


# speedup vs baseline: 1.0559x; 1.0197x over previous
import jax
import jax.numpy as jnp
from jax import lax
from jax.experimental import pallas as pl
from jax.experimental.pallas import tpu as pltpu
from jax.experimental.pallas import tpu_sc as plsc

N_NODES = 10000
N_EDGES = 320000
D = 128

NC = 2
NS = 16
NW = NC * NS

CHUNK = 128
NCHUNKS = 80
EPW = NCHUNKS * CHUNK
E_PAD = EPW * NW

N_PAD = 10240
ROWS_PER_TILE = N_PAD // NS

_mesh = plsc.VectorSubcoreMesh(core_axis_name="c", subcore_axis_name="s")


def _deg_body(ei_hbm, out_hbm, idx_v, ones_v, zb_v, acc_sh, sem):
    c = lax.axis_index("c")
    s = lax.axis_index("s")
    wid = s * NC + c
    for i in range(CHUNK // 16):
        ones_v[pl.ds(16 * i, 16)] = jnp.full((16,), 1.0, jnp.float32)
        zb_v[pl.ds(16 * i, 16)] = jnp.zeros((16,), jnp.float32)
    for k in range(ROWS_PER_TILE // CHUNK):
        pltpu.sync_copy(zb_v, acc_sh.at[pl.ds(s * ROWS_PER_TILE + k * CHUNK, CHUNK)])
    pltpu.sync_copy(ei_hbm.at[wid], idx_v)
    plsc.subcore_barrier()

    pltpu.async_copy(ones_v, acc_sh.at[idx_v.at[0, 1]], sem[0], add=True)
    pltpu.async_copy(ones_v, acc_sh.at[idx_v.at[1, 1]], sem[1], add=True)

    @pl.loop(2, NCHUNKS, step=2)
    def _(j):
        pltpu.make_async_copy(ones_v, acc_sh.at[idx_v.at[j, 1]], sem[0]).wait()
        pltpu.async_copy(ones_v, acc_sh.at[idx_v.at[j, 1]], sem[0], add=True)
        pltpu.make_async_copy(ones_v, acc_sh.at[idx_v.at[j + 1, 1]], sem[1]).wait()
        pltpu.async_copy(ones_v, acc_sh.at[idx_v.at[j + 1, 1]], sem[1], add=True)

    pltpu.make_async_copy(ones_v, acc_sh.at[idx_v.at[0, 1]], sem[0]).wait()
    pltpu.make_async_copy(ones_v, acc_sh.at[idx_v.at[1, 1]], sem[1]).wait()

    plsc.subcore_barrier()
    pltpu.sync_copy(acc_sh.at[pl.ds(s * ROWS_PER_TILE, ROWS_PER_TILE)],
                    out_hbm.at[c, pl.ds(s * ROWS_PER_TILE, ROWS_PER_TILE)])


def _deg_counts(edge_idx):
    k = pl.kernel(
        _deg_body,
        out_type=jax.ShapeDtypeStruct((NC, N_PAD), jnp.float32),
        mesh=_mesh,
        scratch_types=[
            pltpu.VMEM((NCHUNKS + 2, 2, CHUNK), jnp.int32),
            pltpu.VMEM((CHUNK,), jnp.float32),
            pltpu.VMEM((CHUNK,), jnp.float32),
            pltpu.VMEM_SHARED((N_PAD,), jnp.float32),
            [pltpu.SemaphoreType.DMA for _ in range(2)],
        ],
    )
    return k(edge_idx)


def _scat_body(y_hbm, ei_hbm, out_hbm,
               bufs, idxs, acc_sh, gs, ss, isem):
    c = lax.axis_index("c")
    s = lax.axis_index("s")
    wid = s * NC + c
    rows = pl.ds(s * ROWS_PER_TILE, ROWS_PER_TILE)

    @pl.when(c == 0)
    def _():
        pltpu.sync_copy(y_hbm.at[rows], acc_sh.at[rows])

    @pl.when(c == 1)
    def _():
        @pl.loop(0, CHUNK)
        def _(r):
            for i in range(D // 16):
                bufs[0].at[r][pl.ds(16 * i, 16)] = jnp.zeros((16,), jnp.float32)
        for k in range(ROWS_PER_TILE // CHUNK):
            pltpu.sync_copy(
                bufs[0],
                acc_sh.at[pl.ds(s * ROWS_PER_TILE + k * CHUNK, CHUNK)])

    def load_idx(j, q):
        pltpu.async_copy(ei_hbm.at[wid, j], idxs[q], isem[q])

    def wait_idx(q):
        pltpu.make_async_copy(ei_hbm.at[wid, 0], idxs[q], isem[q]).wait()

    def step(j, b, q, first):
        if not first:
            pltpu.make_async_copy(bufs[b], acc_sh.at[idxs[q].at[1]], ss[b]).wait()
        load_idx(j + 2, (q + 2) % 4)
        wait_idx(q)
        pltpu.async_copy(y_hbm.at[idxs[q].at[0]], bufs[b], gs[b]).wait()
        pltpu.async_copy(bufs[b], acc_sh.at[idxs[q].at[1]], ss[b], add=True)

    plsc.subcore_barrier()
    load_idx(0, 0)
    load_idx(1, 1)
    step(0, 0, 0, True)
    step(1, 1, 1, True)
    step(2, 0, 2, False)
    step(3, 1, 3, False)

    @pl.loop(4, NCHUNKS, step=4)
    def _(j):
        step(j, 0, 0, False)
        step(j + 1, 1, 1, False)
        step(j + 2, 0, 2, False)
        step(j + 3, 1, 3, False)

    pltpu.make_async_copy(bufs[0], acc_sh.at[idxs[0].at[1]], ss[0]).wait()
    pltpu.make_async_copy(bufs[1], acc_sh.at[idxs[1].at[1]], ss[1]).wait()
    wait_idx(NCHUNKS % 4)
    wait_idx((NCHUNKS + 1) % 4)

    plsc.subcore_barrier()
    pltpu.sync_copy(acc_sh.at[pl.ds(s * ROWS_PER_TILE, ROWS_PER_TILE)],
                    out_hbm.at[c, pl.ds(s * ROWS_PER_TILE, ROWS_PER_TILE)])


def _edge_scatter(y, edge_idx):
    k = pl.kernel(
        _scat_body,
        out_type=jax.ShapeDtypeStruct((NC, N_PAD, D), jnp.float32),
        mesh=_mesh,
        scratch_types=[
            [pltpu.VMEM((CHUNK, D), jnp.float32) for _ in range(2)],
            [pltpu.VMEM((2, CHUNK), jnp.int32) for _ in range(4)],
            pltpu.VMEM_SHARED((N_PAD, D), jnp.float32),
            [pltpu.SemaphoreType.DMA for _ in range(2)],
            [pltpu.SemaphoreType.DMA for _ in range(2)],
            [pltpu.SemaphoreType.DMA for _ in range(4)],
        ],
    )
    return k(y, edge_idx)


BLK = 512
GRID = N_PAD // BLK


def _t1_body(ca_ref, cb_ref, x_ref, w_ref, y_ref, dinv_ref):
    deg = 1.0 + ca_ref[...] + cb_ref[...]
    dinv = lax.rsqrt(deg)
    dinv_ref[...] = dinv
    xw = jnp.dot(x_ref[...], w_ref[...], preferred_element_type=jnp.float32)
    y_ref[...] = xw * dinv


def _tc_first(cnt_a, cnt_b, x_pad, W1):
    return pl.pallas_call(
        _t1_body,
        grid=(GRID,),
        in_specs=[
            pl.BlockSpec((BLK, 1), lambda i: (i, 0)),
            pl.BlockSpec((BLK, 1), lambda i: (i, 0)),
            pl.BlockSpec((BLK, D), lambda i: (i, 0)),
            pl.BlockSpec((D, D), lambda i: (0, 0)),
        ],
        out_specs=[
            pl.BlockSpec((BLK, D), lambda i: (i, 0)),
            pl.BlockSpec((BLK, 1), lambda i: (i, 0)),
        ],
        out_shape=[
            jax.ShapeDtypeStruct((N_PAD, D), jnp.float32),
            jax.ShapeDtypeStruct((N_PAD, 1), jnp.float32),
        ],
    )(cnt_a, cnt_b, x_pad, W1)


def _t2_body(sa_ref, sb_ref, dinv_ref, b_ref, w_ref, y2_ref):
    dinv = dinv_ref[...]
    agg = sa_ref[...] + sb_ref[...]
    h = jnp.maximum(agg * dinv + b_ref[...], 0.0)
    y2_ref[...] = jnp.dot(h, w_ref[...], preferred_element_type=jnp.float32) * dinv


def _tc_mid(sa, sb, dinv, b, W):
    return pl.pallas_call(
        _t2_body,
        grid=(GRID,),
        in_specs=[
            pl.BlockSpec((BLK, D), lambda i: (i, 0)),
            pl.BlockSpec((BLK, D), lambda i: (i, 0)),
            pl.BlockSpec((BLK, 1), lambda i: (i, 0)),
            pl.BlockSpec((1, D), lambda i: (0, 0)),
            pl.BlockSpec((D, D), lambda i: (0, 0)),
        ],
        out_specs=pl.BlockSpec((BLK, D), lambda i: (i, 0)),
        out_shape=jax.ShapeDtypeStruct((N_PAD, D), jnp.float32),
    )(sa, sb, dinv, b, W)


def _t3_body(sa_ref, sb_ref, dinv_ref, b_ref, w_ref, bo_ref, o_ref):
    agg = sa_ref[...] + sb_ref[...]
    h = jnp.maximum(agg * dinv_ref[...] + b_ref[...], 0.0)
    o_ref[...] = jnp.dot(h, w_ref[...], preferred_element_type=jnp.float32) + bo_ref[...]


def _tc_last(sa, sb, dinv, b, W, bo):
    return pl.pallas_call(
        _t3_body,
        grid=(GRID,),
        in_specs=[
            pl.BlockSpec((BLK, D), lambda i: (i, 0)),
            pl.BlockSpec((BLK, D), lambda i: (i, 0)),
            pl.BlockSpec((BLK, 1), lambda i: (i, 0)),
            pl.BlockSpec((1, D), lambda i: (0, 0)),
            pl.BlockSpec((D, D), lambda i: (0, 0)),
            pl.BlockSpec((1, D), lambda i: (0, 0)),
        ],
        out_specs=pl.BlockSpec((BLK, D), lambda i: (i, 0)),
        out_shape=jax.ShapeDtypeStruct((N_NODES, D), jnp.float32),
    )(sa, sb, dinv, b, W, bo)


def kernel(x, edge_index, W1, b1, W2, b2, Wo, bo):
    src = edge_index[0].astype(jnp.int32)
    dst = edge_index[1].astype(jnp.int32)

    n_fill = E_PAD - N_EDGES
    fill = (N_NODES + (jnp.arange(n_fill, dtype=jnp.int32) % (N_PAD - N_NODES)))
    src_p = jnp.concatenate([src, fill]).reshape(NW, NCHUNKS, CHUNK)
    dst_p = jnp.concatenate([dst, fill]).reshape(NW, NCHUNKS, CHUNK)
    ei = jnp.stack([src_p, dst_p], axis=2)
    ei = jnp.concatenate(
        [ei, jnp.zeros((NW, 2, 2, CHUNK), jnp.int32)], axis=1)

    x_pad = jnp.zeros((N_PAD, D), jnp.float32).at[:N_NODES].set(x)

    cnt = _deg_counts(ei)
    cnt_a = cnt[0].reshape(N_PAD, 1)
    cnt_b = cnt[1].reshape(N_PAD, 1)

    y1, dinv = _tc_first(cnt_a, cnt_b, x_pad, W1)
    s1 = _edge_scatter(y1, ei)
    y2 = _tc_mid(s1[0], s1[1], dinv, b1.reshape(1, D), W2)
    s2 = _edge_scatter(y2, ei)
    return _tc_last(s2[0], s2[1], dinv, b2.reshape(1, D), Wo, bo.reshape(1, D))
